# baseline (speedup 1.0000x reference)
import functools
import jax
import jax.numpy as jnp
from jax import lax
from jax.experimental import pallas as pl
from jax.experimental.pallas import tpu as pltpu
from jax.experimental.pallas import tpu_sc as plsc

_E = 64
_D = 8
_F = 16
_NW = 32
_TILE = 8192


def _router_tile_kernel(x_ref, gwt_ref, wsgu_ref, wsdt_ref, info_ref, sh_ref):
    f32 = jnp.float32
    x = x_ref[...]

    logits = jnp.dot(x, gwt_ref[...], preferred_element_type=f32)
    m = jnp.max(logits, axis=-1, keepdims=True)
    w = 1.0 / jnp.sum(jnp.exp(logits - m), axis=-1, keepdims=True)
    lane = jax.lax.broadcasted_iota(jnp.int32, logits.shape, 1)
    am = jnp.min(jnp.where(logits == m, lane, _E), axis=-1, keepdims=True)
    info_ref[...] = jnp.concatenate(
        [am.astype(f32), w, jnp.zeros((x.shape[0], 6), f32)], axis=1)

    gu = jnp.dot(x, wsgu_ref[...], preferred_element_type=f32)
    gs = gu[:, :_F]
    us = gu[:, _F:]
    hs = (gs * jax.nn.sigmoid(gs)) * us
    sh_ref[...] = jnp.dot(hs, wsdt_ref[...], preferred_element_type=f32)


def _sc_dispatch_body(c_tok, xw_hbm, wall_hbm, ri_hbm, sh_hbm, out_hbm,
                      xv, wv, riv, shv, obuf, hbuf, sem):
    wid = lax.axis_index("s") * 2 + lax.axis_index("c")
    pltpu.sync_copy(xw_hbm.at[wid], xv)
    pltpu.sync_copy(wall_hbm, wv)
    pltpu.sync_copy(ri_hbm.at[wid], riv)
    pltpu.sync_copy(sh_hbm.at[wid], shv)

    iota16 = lax.iota(jnp.int32, 16)
    n_groups = c_tok // 16

    def group_body(gi, carry):
        tbase = gi * 16
        row = (tbase + iota16) * _D
        xd = [plsc.load_gather(xv, [d * c_tok + tbase + iota16])
              for d in range(_D)]
        sel = plsc.load_gather(riv, [row]).astype(jnp.int32)
        wgt = plsc.load_gather(riv, [row + 1])
        base = sel * (3 * _D * _F)

        for f in range(_F):
            g = xd[0] * plsc.load_gather(wv, [base + f])
            u = xd[0] * plsc.load_gather(wv, [base + 128 + f])
            for d in range(1, _D):
                off = d * _F + f
                g = g + xd[d] * plsc.load_gather(wv, [base + off])
                u = u + xd[d] * plsc.load_gather(wv, [base + 128 + off])
            sg = 1.0 / (1.0 + jnp.exp(-g))
            hbuf[pl.ds(f * 16, 16)] = g * sg * u * wgt

        for d in range(_D):
            doff = 256 + d * _F
            o = hbuf[pl.ds(0, 16)] * plsc.load_gather(wv, [base + doff])
            for f in range(1, _F):
                o = o + hbuf[pl.ds(f * 16, 16)] * plsc.load_gather(
                    wv, [base + doff + f])
            o = o + plsc.load_gather(shv, [row + d])
            plsc.store_scatter(obuf, [row + d], o)
        return carry

    lax.fori_loop(0, n_groups, group_body, 0)
    pltpu.sync_copy(obuf, out_hbm.at[wid])


def kernel(hidden_states, gate_weight, Wg, Wu, Wd, Wsg, Wsu, Wsd):
    orig_shape = hidden_states.shape
    x2 = hidden_states.reshape(-1, _D)
    t = x2.shape[0]

    wg_p = Wg.transpose(0, 2, 1).reshape(_E, _D * _F)
    wu_p = Wu.transpose(0, 2, 1).reshape(_E, _D * _F)
    wd_p = Wd.reshape(_E, _D * _F)
    w_all = jnp.concatenate([wg_p, wu_p, wd_p], axis=1)
    wsgu = jnp.concatenate([Wsg.T, Wsu.T], axis=1)

    grid = (t // _TILE,)
    zero = lambda i: (0, 0)
    info, shared = pl.pallas_call(
        _router_tile_kernel,
        grid=grid,
        in_specs=[
            pl.BlockSpec((_TILE, _D), lambda i: (i, 0)),
            pl.BlockSpec((_D, _E), zero),
            pl.BlockSpec((_D, 2 * _F), zero),
            pl.BlockSpec((_F, _D), zero),
        ],
        out_specs=[
            pl.BlockSpec((_TILE, _D), lambda i: (i, 0)),
            pl.BlockSpec((_TILE, _D), lambda i: (i, 0)),
        ],
        out_shape=[
            jax.ShapeDtypeStruct((t, _D), jnp.float32),
            jax.ShapeDtypeStruct((t, _D), jnp.float32),
        ],
    )(x2, gate_weight.T, wsgu, Wsd.T)

    c_tok = t // _NW
    xw = x2.reshape(_NW, c_tok, _D).transpose(0, 2, 1).reshape(_NW, _D * c_tok)
    mesh = plsc.VectorSubcoreMesh(core_axis_name="c", subcore_axis_name="s")
    f = pl.kernel(
        functools.partial(_sc_dispatch_body, c_tok),
        out_type=jax.ShapeDtypeStruct((_NW, c_tok * _D), jnp.float32),
        mesh=mesh,
        scratch_types=[
            pltpu.VMEM((_D * c_tok,), jnp.float32),
            pltpu.VMEM((_E * 3 * _D * _F,), jnp.float32),
            pltpu.VMEM((c_tok * _D,), jnp.float32),
            pltpu.VMEM((c_tok * _D,), jnp.float32),
            pltpu.VMEM((c_tok * _D,), jnp.float32),
            pltpu.VMEM((_F * 16,), jnp.float32),
            pltpu.SemaphoreType.DMA,
        ],
        compiler_params=pltpu.CompilerParams(needs_layout_passes=False),
    )
    out = f(xw, w_all.reshape(-1), info.reshape(_NW, c_tok * _D),
            shared.reshape(_NW, c_tok * _D))
    return out.reshape(t, _D).reshape(orig_shape)
